# Initial kernel scaffold; baseline (speedup 1.0000x reference)
import jax
import jax.numpy as jnp
from jax.experimental import pallas as pl


def kernel(h_V, h_P, P_idx, batch_id, W_ne, b_ne, W_ee, b_ee, g_nn, b_nn, g_en, b_en, Wv1, bv1, Wv2, bv2, Wv3, bv3, gv1, bnv1, gv2, bnv2, W_we, b_we, Wm1, bm1, Wm2, bm2, Wm3, bm3, Wa, ba, gl1, bl1, Wf1, bf1, Wf2, bf2, gl2, bl2, We1, be1, We2, be2, gle, ble, Wout, bout):
    raise NotImplementedError("write your pallas kernel here")


# speedup vs baseline: 1.6508x; 1.6508x over previous
import functools

import jax
import jax.numpy as jnp
from jax import lax
from jax.experimental import pallas as pl
from jax.experimental.pallas import tpu as pltpu
from jax.experimental.pallas import tpu_sc as plsc

H = 128
L = 3
NHEAD = 4
EPS = 1e-5
N_NODES = 10000
N_EDGES = 160000

EB = 2000
NB = 2000

_INTERPRET = False


def _full(shape):
    return pl.BlockSpec(shape, lambda i: tuple(0 for _ in shape))


def _blk(rows, cols):
    return pl.BlockSpec((rows, cols), lambda i: (i, 0))


def _ln(v, g, b):
    mu = jnp.mean(v, axis=-1, keepdims=True)
    d = v - mu
    var = jnp.mean(d * d, axis=-1, keepdims=True)
    return d * lax.rsqrt(var + EPS) * g + b


def _bn_affine(st, g, b, n):
    m = st[0:1, :] / n
    var = st[1:2, :] / n - m * m
    a = g * lax.rsqrt(var + EPS)
    return a, b - m * a


def _sel16():
    r = lax.broadcasted_iota(jnp.int32, (16, H), 0)
    c = lax.broadcasted_iota(jnp.int32, (16, H), 1)
    return (c // (H // NHEAD) == r).astype(jnp.float32)


def _mm_stats_body(x_ref, w_ref, b_ref, out_ref, st_ref):
    y = jnp.dot(x_ref[...], w_ref[...]) + b_ref[...]
    out_ref[...] = y

    @pl.when(pl.program_id(0) == 0)
    def _():
        st_ref[...] = jnp.zeros_like(st_ref)

    st_ref[0:1, :] += jnp.sum(y, axis=0, keepdims=True)
    st_ref[1:2, :] += jnp.sum(y * y, axis=0, keepdims=True)


def _mm_stats(x, w, b, rows, blk):
    n, k = x.shape
    h = w.shape[1]
    return pl.pallas_call(
        _mm_stats_body,
        grid=(n // blk,),
        in_specs=[_blk(blk, k), _full(w.shape), _full(b.shape)],
        out_specs=[_blk(blk, h), _full((8, h))],
        out_shape=[jax.ShapeDtypeStruct((n, h), jnp.float32),
                   jax.ShapeDtypeStruct((8, h), jnp.float32)],
        interpret=_INTERPRET,
    )(x, w, b)


def _bn_mm_body(gelu_out, want_stats, nrows, x_ref, st_in_ref, g_ref, bb_ref,
                w_ref, b_ref, out_ref, *st_out):
    a, c = _bn_affine(st_in_ref[...], g_ref[...], bb_ref[...], nrows)
    y = jnp.dot(x_ref[...] * a + c, w_ref[...]) + b_ref[...]
    if gelu_out:
        y = jax.nn.gelu(y)
    out_ref[...] = y
    if want_stats:
        st_ref = st_out[0]

        @pl.when(pl.program_id(0) == 0)
        def _():
            st_ref[...] = jnp.zeros_like(st_ref)

        st_ref[0:1, :] += jnp.sum(y, axis=0, keepdims=True)
        st_ref[1:2, :] += jnp.sum(y * y, axis=0, keepdims=True)


def _bn_mm(x, st, g, bb, w, b, gelu_out, want_stats, blk):
    n, k = x.shape
    h = w.shape[1]
    out_specs = [_blk(blk, h)]
    out_shape = [jax.ShapeDtypeStruct((n, h), jnp.float32)]
    if want_stats:
        out_specs.append(_full((8, h)))
        out_shape.append(jax.ShapeDtypeStruct((8, h), jnp.float32))
    r = pl.pallas_call(
        functools.partial(_bn_mm_body, gelu_out, want_stats, float(n)),
        grid=(n // blk,),
        in_specs=[_blk(blk, k), _full((8, k)), _full(g.shape), _full(bb.shape),
                  _full(w.shape), _full(b.shape)],
        out_specs=out_specs,
        out_shape=out_shape,
        interpret=_INTERPRET,
    )(x, st, g, bb, w, b)
    return r if want_stats else r[0]


def _msg_body(xs_ref, xd_ref, e_ref, wm1_ref, bm1_ref, wm2_ref, bm2_ref,
              wm3_ref, bm3_ref, wa_ref, ba_ref, z_ref, ex_ref):
    xs, xd, e = xs_ref[...], xd_ref[...], e_ref[...]
    wm1 = wm1_ref[...]
    h = (jnp.dot(xs, wm1[0:H]) + jnp.dot(e, wm1[H:2 * H])
         + jnp.dot(xd, wm1[2 * H:3 * H]) + bm1_ref[...])
    m = jax.nn.gelu(h)
    m = jax.nn.gelu(jnp.dot(m, wm2_ref[...]) + bm2_ref[...])
    m = jnp.dot(m, wm3_ref[...]) + bm3_ref[...]
    wa = wa_ref[...]
    lg = (jnp.dot(xs, wa[0:H]) + jnp.dot(e, wa[H:2 * H])
          + jnp.dot(xd, wa[2 * H:3 * H]) + ba_ref[...])
    ex = jnp.exp(lg)
    ex_ref[...] = ex
    z_ref[...] = m * jnp.dot(ex, _sel16())


def _msg(xs, xd, e, wm1, bm1, wm2, bm2, wm3, bm3, wa16, ba16):
    return pl.pallas_call(
        _msg_body,
        grid=(N_EDGES // EB,),
        in_specs=[_blk(EB, H), _blk(EB, H), _blk(EB, H),
                  _full((3 * H, H)), _full((1, H)), _full((H, H)),
                  _full((1, H)), _full((H, H)), _full((1, H)),
                  _full((3 * H, 16)), _full((1, 16))],
        out_specs=[_blk(EB, H), _blk(EB, 16)],
        out_shape=[jax.ShapeDtypeStruct((N_EDGES, H), jnp.float32),
                   jax.ShapeDtypeStruct((N_EDGES, 16), jnp.float32)],
        interpret=_INTERPRET,
    )(xs, xd, e, wm1, bm1, wm2, bm2, wm3, bm3, wa16, ba16)


def _node_body(x_ref, zp_ref, dp_ref, gl1_ref, bl1_ref, wf1_ref, bf1_ref,
               wf2_ref, bf2_ref, gl2_ref, bl2_ref, out_ref):
    z = zp_ref[0] + zp_ref[1]
    den = dp_ref[0] + dp_ref[1]
    inv = 1.0 / (den + 1e-9)
    dh = z * jnp.dot(inv, _sel16())
    x1 = _ln(x_ref[...] + dh, gl1_ref[...], bl1_ref[...])
    ff = jnp.dot(jax.nn.gelu(jnp.dot(x1, wf1_ref[...]) + bf1_ref[...]),
                 wf2_ref[...]) + bf2_ref[...]
    out_ref[...] = _ln(x1 + ff, gl2_ref[...], bl2_ref[...])


def _node(x, zp, dp, gl1, bl1, wf1, bf1, wf2, bf2, gl2, bl2):
    return pl.pallas_call(
        _node_body,
        grid=(N_NODES // NB,),
        in_specs=[_blk(NB, H),
                  pl.BlockSpec((2, NB, H), lambda i: (0, i, 0)),
                  pl.BlockSpec((2, NB, 16), lambda i: (0, i, 0)),
                  _full((1, H)), _full((1, H)), _full((H, 4 * H)),
                  _full((1, 4 * H)), _full((4 * H, H)), _full((1, H)),
                  _full((1, H)), _full((1, H))],
        out_specs=_blk(NB, H),
        out_shape=jax.ShapeDtypeStruct((N_NODES, H), jnp.float32),
        interpret=_INTERPRET,
    )(x, zp, dp, gl1, bl1, wf1, bf1, wf2, bf2, gl2, bl2)


def _eupd_body(xs_ref, xd_ref, e_ref, we1_ref, be1_ref, we2_ref, be2_ref,
               gle_ref, ble_ref, out_ref):
    e = e_ref[...]
    we1 = we1_ref[...]
    h = (jnp.dot(xs_ref[...], we1[0:H]) + jnp.dot(e, we1[H:2 * H])
         + jnp.dot(xd_ref[...], we1[2 * H:3 * H]) + be1_ref[...])
    de = jnp.dot(jax.nn.gelu(h), we2_ref[...]) + be2_ref[...]
    out_ref[...] = _ln(e + de, gle_ref[...], ble_ref[...])


def _eupd(xs, xd, e, we1, be1, we2, be2, gle, ble):
    return pl.pallas_call(
        _eupd_body,
        grid=(N_EDGES // EB,),
        in_specs=[_blk(EB, H), _blk(EB, H), _blk(EB, H),
                  _full((3 * H, H)), _full((1, H)), _full((H, H)),
                  _full((1, H)), _full((1, H)), _full((1, H))],
        out_specs=_blk(EB, H),
        out_shape=jax.ShapeDtypeStruct((N_EDGES, H), jnp.float32),
        interpret=_INTERPRET,
    )(xs, xd, e, we1, be1, we2, be2, gle, ble)


def _out_body(x_ref, w_ref, b_ref, out_ref):
    lp = jnp.dot(x_ref[...], w_ref[...]) + b_ref[...]
    col = lax.broadcasted_iota(jnp.int32, lp.shape, 1)
    lp = jnp.where(col < 33, lp, -1e30)
    mx = jnp.max(lp, axis=-1, keepdims=True)
    lse = jnp.log(jnp.sum(jnp.exp(lp - mx), axis=-1, keepdims=True))
    out_ref[...] = (lp - mx - lse)[:, :33]


def _out(x, w64, b64):
    return pl.pallas_call(
        _out_body,
        grid=(N_NODES // NB,),
        in_specs=[_blk(NB, H), _full((H, 64)), _full((1, 64))],
        out_specs=_blk(NB, 33),
        out_shape=jax.ShapeDtypeStruct((N_NODES, 33), jnp.float32),
        interpret=_INTERPRET,
    )(x, w64, b64)


def _gather(x, src, dst):
    return x[src], x[dst]


def _scatter(z, ex, dst):
    zp = jax.ops.segment_sum(z, dst, num_segments=N_NODES)
    dp = jax.ops.segment_sum(ex, dst, num_segments=N_NODES)
    return (jnp.stack([zp, jnp.zeros_like(zp)]),
            jnp.stack([dp, jnp.zeros_like(dp)]))


def kernel(h_V, h_P, P_idx, batch_id, W_ne, b_ne, W_ee, b_ee, g_nn, b_nn,
           g_en, b_en, Wv1, bv1, Wv2, bv2, Wv3, bv3, gv1, bnv1, gv2, bnv2,
           W_we, b_we, Wm1, bm1, Wm2, bm2, Wm3, bm3, Wa, ba, gl1, bl1,
           Wf1, bf1, Wf2, bf2, gl2, bl2, We1, be1, We2, be2, gle, ble,
           Wout, bout):
    row = lambda v: v.reshape(1, -1)
    src, dst = P_idx[0], P_idx[1]

    y1, st1 = _mm_stats(h_V, W_ne, row(b_ne), N_NODES, NB)
    y2, st2 = _bn_mm(y1, st1, row(g_nn), row(b_nn), Wv1, row(bv1),
                     True, True, NB)
    y3, st3 = _bn_mm(y2, st2, row(gv1), row(bnv1), Wv2, row(bv2),
                     True, True, NB)
    x = _bn_mm(y3, st3, row(gv2), row(bnv2), Wv3, row(bv3), False, False, NB)

    t, ste = _mm_stats(h_P, W_ee, row(b_ee), N_EDGES, EB)
    e = _bn_mm(t, ste, row(g_en), row(b_en), W_we, row(b_we), False, False, EB)

    for l in range(L):
        wa16 = jnp.zeros((3 * H, 16), jnp.float32).at[:, :NHEAD].set(Wa[l])
        ba16 = jnp.zeros((1, 16), jnp.float32).at[:, :NHEAD].set(ba[l])
        xs, xd = _gather(x, src, dst)
        z, ex = _msg(xs, xd, e, Wm1[l], row(bm1[l]), Wm2[l], row(bm2[l]),
                     Wm3[l], row(bm3[l]), wa16, ba16)
        zp, dp = _scatter(z, ex, dst)
        x = _node(x, zp, dp, row(gl1[l]), row(bl1[l]), Wf1[l], row(bf1[l]),
                  Wf2[l], row(bf2[l]), row(gl2[l]), row(bl2[l]))
        if l < L - 1:
            xs2, xd2 = _gather(x, src, dst)
            e = _eupd(xs2, xd2, e, We1[l], row(be1[l]), We2[l], row(be2[l]),
                      row(gle[l]), row(ble[l]))

    w64 = jnp.zeros((H, 64), jnp.float32).at[:, :33].set(Wout)
    b64 = jnp.zeros((1, 64), jnp.float32).at[:, :33].set(bout)
    return _out(x, w64, b64)
